# baseline (speedup 1.0000x reference)
import functools

import jax
import jax.numpy as jnp
from jax import lax
from jax.experimental import pallas as pl
from jax.experimental.pallas import tpu as pltpu
from jax.experimental.pallas import tpu_sc as plsc

NUM_EMB = 1_000_000
DIM = 64
BATCH = 16384
FIELDS = 100

NUM_CORES = 2
NUM_SUBCORES = 16
NW = NUM_CORES * NUM_SUBCORES
B_PER_W = BATCH // NW
SG = 8
SB = 4
N_GROUPS = B_PER_W // SG
PR = 104


def _sc_gather(x, table_wide):
    mesh = plsc.VectorSubcoreMesh(core_axis_name="c", subcore_axis_name="s")

    @functools.partial(
        pl.kernel,
        mesh=mesh,
        out_type=jax.ShapeDtypeStruct((BATCH, FIELDS, DIM), jnp.float32),
        scratch_types=[
            pltpu.VMEM((SG, FIELDS), jnp.int32),
            pltpu.VMEM((SB * PR, 2 * DIM), jnp.float32),
            pltpu.VMEM((SB, FIELDS, DIM), jnp.float32),
            pltpu.SemaphoreType.DMA,
        ],
    )
    def k(x_hbm, table_hbm, out_hbm, idx_v, pair_v, stage_v, sem):
        wid = lax.axis_index("s") * NUM_CORES + lax.axis_index("c")
        base = wid * B_PER_W

        def group(g, carry):
            b0 = pl.multiple_of(base + g * SG, SG)
            pltpu.sync_copy(x_hbm.at[pl.ds(b0, SG)], idx_v)

            for s in range(SG // SB):
                copies = []
                for j in range(SB):
                    copies.append(
                        pltpu.async_copy(
                            table_hbm.at[idx_v.at[s * SB + j]],
                            pair_v.at[pl.ds(j * PR, FIELDS), :],
                            sem,
                        )
                    )
                for c in copies:
                    c.wait()

                for j in range(SB):
                    def depad(f, carry2):
                        for kk in range(DIM // 16):
                            stage_v[j, f, pl.ds(kk * 16, 16)] = pair_v[
                                j * PR + f, pl.ds(kk * 16, 16)
                            ]
                        return carry2

                    lax.fori_loop(0, FIELDS, depad, 0)

                pltpu.sync_copy(stage_v, out_hbm.at[pl.ds(b0 + s * SB, SB)])
            return carry

        lax.fori_loop(0, N_GROUPS, group, 0)

    return k(x, table_wide)


@jax.jit
def kernel(x, weight):
    table_wide = jnp.pad(weight, ((0, 0), (0, DIM)))
    return _sc_gather(x, table_wide)


# speedup vs baseline: 1.1335x; 1.1335x over previous
import functools

import jax
import jax.numpy as jnp
from jax import lax
from jax.experimental import pallas as pl
from jax.experimental.pallas import tpu as pltpu
from jax.experimental.pallas import tpu_sc as plsc

NUM_EMB = 1_000_000
DIM = 64
BATCH = 16384
FIELDS = 100

NUM_CORES = 2
NUM_SUBCORES = 16
NW = NUM_CORES * NUM_SUBCORES
B_PER_W = BATCH // NW
SG = 8
SB = 2
NS = SG // SB
N_GROUPS = B_PER_W // SG
PR = 104


def _sc_gather(x, table_wide):
    mesh = plsc.VectorSubcoreMesh(core_axis_name="c", subcore_axis_name="s")

    @functools.partial(
        pl.kernel,
        mesh=mesh,
        out_type=jax.ShapeDtypeStruct((BATCH, FIELDS, DIM), jnp.float32),
        scratch_types=[
            pltpu.VMEM((SG, FIELDS), jnp.int32),
            pltpu.VMEM((SB * PR, 2 * DIM), jnp.float32),
            pltpu.VMEM((SB * PR, 2 * DIM), jnp.float32),
            pltpu.VMEM((SB, FIELDS, DIM), jnp.float32),
            pltpu.VMEM((SB, FIELDS, DIM), jnp.float32),
            pltpu.SemaphoreType.DMA,
            pltpu.SemaphoreType.DMA,
            pltpu.SemaphoreType.DMA,
            pltpu.SemaphoreType.DMA,
        ],
    )
    def k(x_hbm, table_hbm, out_hbm, idx_v, pair_a, pair_b, stage_a, stage_b,
          gsem_a, gsem_b, wsem_a, wsem_b):
        wid = lax.axis_index("s") * NUM_CORES + lax.axis_index("c")
        base = wid * B_PER_W
        bufs = [(pair_a, stage_a, gsem_a, wsem_a),
                (pair_b, stage_b, gsem_b, wsem_b)]

        def issue(s, pair_v, gsem):
            for j in range(SB):
                pltpu.async_copy(
                    table_hbm.at[idx_v.at[s * SB + j]],
                    pair_v.at[pl.ds(j * PR, FIELDS), :],
                    gsem,
                )

        def wait_gathers(s, pair_v, gsem):
            for j in range(SB):
                pltpu.make_async_copy(
                    table_hbm.at[idx_v.at[s * SB + j]],
                    pair_v.at[pl.ds(j * PR, FIELDS), :],
                    gsem,
                ).wait()

        def depad(pair_v, stage_v):
            for j in range(SB):
                def body(fi, carry2):
                    for u in range(4):
                        f = fi * 4 + u
                        for kk in range(DIM // 16):
                            stage_v[j, f, pl.ds(kk * 16, 16)] = pair_v[
                                j * PR + f, pl.ds(kk * 16, 16)
                            ]
                    return carry2

                lax.fori_loop(0, FIELDS // 4, body, 0)

        def start_wb(b0, s, stage_v, wsem):
            pltpu.async_copy(
                stage_v, out_hbm.at[pl.ds(b0 + s * SB, SB)], wsem
            )

        def wait_wb(b0, s, stage_v, wsem):
            pltpu.make_async_copy(
                stage_v, out_hbm.at[pl.ds(b0 + s * SB, SB)], wsem
            ).wait()

        def group(g, carry):
            b0 = pl.multiple_of(base + g * SG, SG)
            pltpu.sync_copy(x_hbm.at[pl.ds(b0, SG)], idx_v)
            issue(0, pair_a, gsem_a)
            for s in range(NS):
                pair_v, stage_v, gsem, wsem = bufs[s % 2]
                if s + 1 < NS:
                    pn, _, gs2, _ = bufs[(s + 1) % 2]
                    issue(s + 1, pn, gs2)
                wait_gathers(s, pair_v, gsem)
                if s >= 2:
                    wait_wb(b0, s - 2, stage_v, wsem)
                depad(pair_v, stage_v)
                start_wb(b0, s, stage_v, wsem)
            for s in (NS - 2, NS - 1):
                pair_v, stage_v, gsem, wsem = bufs[s % 2]
                wait_wb(b0, s, stage_v, wsem)
            return carry

        lax.fori_loop(0, N_GROUPS, group, 0)

    return k(x, table_wide)


@jax.jit
def kernel(x, weight):
    table_wide = jnp.pad(weight, ((0, 0), (0, DIM)))
    return _sc_gather(x, table_wide)


# speedup vs baseline: 1.1567x; 1.0205x over previous
import functools

import jax
import jax.numpy as jnp
from jax import lax
from jax.experimental import pallas as pl
from jax.experimental.pallas import tpu as pltpu
from jax.experimental.pallas import tpu_sc as plsc

NUM_EMB = 1_000_000
DIM = 64
BATCH = 16384
FIELDS = 100

NUM_CORES = 2
NUM_SUBCORES = 16
NW = NUM_CORES * NUM_SUBCORES
B_PER_W = BATCH // NW
SG = 8
SB = 2
NS = SG // SB
N_GROUPS = B_PER_W // SG
PR = 104


def _sc_gather(x, table_wide):
    mesh = plsc.VectorSubcoreMesh(core_axis_name="c", subcore_axis_name="s")

    @functools.partial(
        pl.kernel,
        mesh=mesh,
        out_type=jax.ShapeDtypeStruct((BATCH, FIELDS, DIM), jnp.float32),
        scratch_types=[
            pltpu.VMEM((SG, FIELDS), jnp.int32),
            pltpu.VMEM((SG, FIELDS), jnp.int32),
            pltpu.VMEM((SB * PR, 2 * DIM), jnp.float32),
            pltpu.VMEM((SB * PR, 2 * DIM), jnp.float32),
            pltpu.VMEM((SB, FIELDS, DIM), jnp.float32),
            pltpu.VMEM((SB, FIELDS, DIM), jnp.float32),
            pltpu.SemaphoreType.DMA,
            pltpu.SemaphoreType.DMA,
            pltpu.SemaphoreType.DMA,
            pltpu.SemaphoreType.DMA,
            pltpu.SemaphoreType.DMA,
        ],
    )
    def k(x_hbm, table_hbm, out_hbm, idx_a, idx_b, pair_a, pair_b,
          stage_a, stage_b, gsem_a, gsem_b, wsem_a, wsem_b, isem):
        wid = lax.axis_index("s") * NUM_CORES + lax.axis_index("c")
        base = wid * B_PER_W
        bufs = [(pair_a, stage_a, gsem_a, wsem_a),
                (pair_b, stage_b, gsem_b, wsem_b)]
        idx_bufs = [idx_a, idx_b]

        def issue(idx_v, s, pair_v, gsem):
            for j in range(SB):
                pltpu.async_copy(
                    table_hbm.at[idx_v.at[s * SB + j]],
                    pair_v.at[pl.ds(j * PR, FIELDS), :],
                    gsem,
                )

        def wait_gathers(idx_v, s, pair_v, gsem):
            for j in range(SB):
                pltpu.make_async_copy(
                    table_hbm.at[idx_v.at[s * SB + j]],
                    pair_v.at[pl.ds(j * PR, FIELDS), :],
                    gsem,
                ).wait()

        def depad(pair_v, stage_v):
            for j in range(SB):
                def body(fi, carry2):
                    for u in range(4):
                        f = fi * 4 + u
                        for kk in range(DIM // 16):
                            stage_v[j, f, pl.ds(kk * 16, 16)] = pair_v[
                                j * PR + f, pl.ds(kk * 16, 16)
                            ]
                    return carry2

                lax.fori_loop(0, FIELDS // 4, body, 0)

        def start_wb(b0, s, stage_v, wsem):
            pltpu.async_copy(
                stage_v, out_hbm.at[pl.ds(b0 + s * SB, SB)], wsem
            )

        def wait_wb(b0, s, stage_v, wsem):
            pltpu.make_async_copy(
                stage_v, out_hbm.at[pl.ds(b0 + s * SB, SB)], wsem
            ).wait()

        def idx_fetch(g, idx_v):
            b0 = pl.multiple_of(base + g * SG, SG)
            pltpu.async_copy(x_hbm.at[pl.ds(b0, SG)], idx_v, isem)

        def idx_wait(g, idx_v):
            b0 = pl.multiple_of(base + g * SG, SG)
            pltpu.make_async_copy(x_hbm.at[pl.ds(b0, SG)], idx_v, isem).wait()

        def group(g, gi, carry):
            idx_v = idx_bufs[gi]
            b0 = pl.multiple_of(base + g * SG, SG)
            idx_wait(g, idx_v)
            issue(idx_v, 0, pair_a, gsem_a)

            @pl.when(g + 1 < N_GROUPS)
            def _():
                idx_fetch(g + 1, idx_bufs[1 - gi])

            for s in range(NS):
                pair_v, stage_v, gsem, wsem = bufs[s % 2]
                if s + 1 < NS:
                    pn, _, gs2, _ = bufs[(s + 1) % 2]
                    issue(idx_v, s + 1, pn, gs2)
                wait_gathers(idx_v, s, pair_v, gsem)
                if s >= 2:
                    wait_wb(b0, s - 2, stage_v, wsem)
                depad(pair_v, stage_v)
                start_wb(b0, s, stage_v, wsem)
            for s in (NS - 2, NS - 1):
                pair_v, stage_v, gsem, wsem = bufs[s % 2]
                wait_wb(b0, s, stage_v, wsem)
            return carry

        idx_fetch(0, idx_a)

        def group_pair(gp, carry):
            g = 2 * gp
            carry = group(g, 0, carry)
            carry = group(g + 1, 1, carry)
            return carry

        lax.fori_loop(0, N_GROUPS // 2, group_pair, 0)

    return k(x, table_wide)


@jax.jit
def kernel(x, weight):
    table_wide = jnp.pad(weight, ((0, 0), (0, DIM)))
    return _sc_gather(x, table_wide)
